# initial kernel scaffold (unmeasured)
import jax
import jax.numpy as jnp
from jax import lax
from jax.experimental import pallas as pl
from jax.experimental.pallas import tpu as pltpu


def kernel(
    x,
):
    def body(*refs):
        pass

    out_shape = jax.ShapeDtypeStruct(..., jnp.float32)
    return pl.pallas_call(body, out_shape=out_shape)(...)


# baseline (device time: 298538 ns/iter reference)
import jax
import jax.numpy as jnp
from jax import lax
from jax.experimental import pallas as pl
from jax.experimental.pallas import tpu as pltpu

N_DEV = 4


def kernel(x):
    m_per, n = x.shape
    ch = m_per // N_DEV
    n_hops = 2 * (N_DEV - 1)

    def body(x_ref, out_ref, comm_ref, send_sems, recv_sems, credit_sem):
        my = lax.axis_index("i")
        left = (my + N_DEV - 1) % N_DEV
        right = (my + 1) % N_DEV

        barrier_sem = pltpu.get_barrier_semaphore()
        for nbr in (left, right):
            pl.semaphore_signal(
                barrier_sem, inc=1,
                device_id=(nbr,), device_id_type=pl.DeviceIdType.MESH,
            )
        pl.semaphore_wait(barrier_sem, 2)

        comm_ref[0, :, :] = x_ref[pl.ds(my * ch, ch), :]

        for h in range(n_hops):
            send_slot = h % 2
            recv_slot = (h + 1) % 2
            if h >= 1:
                pl.semaphore_wait(credit_sem, 1)
            rdma = pltpu.make_async_remote_copy(
                src_ref=comm_ref.at[send_slot],
                dst_ref=comm_ref.at[recv_slot],
                send_sem=send_sems.at[h],
                recv_sem=recv_sems.at[h],
                device_id=(right,),
                device_id_type=pl.DeviceIdType.MESH,
            )
            rdma.start()
            rdma.wait()
            if h < n_hops - 1:
                pl.semaphore_signal(
                    credit_sem, inc=1,
                    device_id=(left,), device_id_type=pl.DeviceIdType.MESH,
                )
            if h < N_DEV - 1:
                c_in = (my - h - 1) % N_DEV
                acc = comm_ref[recv_slot, :, :] + x_ref[pl.ds(c_in * ch, ch), :]
                comm_ref[recv_slot, :, :] = acc
                if h == N_DEV - 2:
                    out_ref[pl.ds(c_in * ch, ch), :] = acc
            else:
                t = h - (N_DEV - 1)
                g = (my - t) % N_DEV
                out_ref[pl.ds(g * ch, ch), :] = comm_ref[recv_slot, :, :]

    return pl.pallas_call(
        body,
        out_shape=jax.ShapeDtypeStruct((m_per, n), x.dtype),
        in_specs=[pl.BlockSpec(memory_space=pltpu.VMEM)],
        out_specs=pl.BlockSpec(memory_space=pltpu.VMEM),
        scratch_shapes=[
            pltpu.VMEM((2, ch, n), x.dtype),
            pltpu.SemaphoreType.DMA((n_hops,)),
            pltpu.SemaphoreType.DMA((n_hops,)),
            pltpu.SemaphoreType.REGULAR,
        ],
        compiler_params=pltpu.CompilerParams(collective_id=0),
    )(x)


# device time: 157061 ns/iter; 1.9008x vs baseline; 1.9008x over previous
import jax
import jax.numpy as jnp
from jax import lax
from jax.experimental import pallas as pl
from jax.experimental.pallas import tpu as pltpu

N_DEV = 4


def kernel(x):
    m, n = x.shape
    hc = n // 2
    hb = m // 2
    qb = m // 4

    def body(x_ref, out_ref, rP1, rP2, rQ1, rQ2, send_sems, recv_sems):
        my = lax.axis_index("i")
        xv = jnp.where(my >= 2, 1, 0)
        yv = jnp.where((my == 1) | (my == 2), 1, 0)
        px = 3 - my
        py = my ^ 1

        barrier_sem = pltpu.get_barrier_semaphore()
        for nbr in (px, py):
            pl.semaphore_signal(
                barrier_sem, inc=1,
                device_id=(nbr,), device_id_type=pl.DeviceIdType.MESH,
            )
        pl.semaphore_wait(barrier_sem, 2)

        bx = xv * hb
        bxp = (1 - xv) * hb
        r0P = bx + yv * qb
        rqP = bx + (1 - yv) * qb

        by = yv * hb
        byp = (1 - yv) * hb
        r0Q = by + xv * qb
        rqQ = by + (1 - xv) * qb

        def xchg(step, src, dst, peer):
            rdma = pltpu.make_async_remote_copy(
                src_ref=src,
                dst_ref=dst,
                send_sem=send_sems.at[step],
                recv_sem=recv_sems.at[step],
                device_id=(peer,),
                device_id_type=pl.DeviceIdType.MESH,
            )
            rdma.start()
            return rdma

        a = xchg(0, x_ref.at[pl.ds(bxp, hb), pl.ds(0, hc)], rP1, px)
        b = xchg(1, x_ref.at[pl.ds(byp, hb), pl.ds(hc, hc)], rQ1, py)
        a.wait()
        b.wait()
        out_ref[pl.ds(bx, hb), pl.ds(0, hc)] = (
            x_ref[pl.ds(bx, hb), pl.ds(0, hc)] + rP1[...]
        )
        out_ref[pl.ds(by, hb), pl.ds(hc, hc)] = (
            x_ref[pl.ds(by, hb), pl.ds(hc, hc)] + rQ1[...]
        )

        a = xchg(2, out_ref.at[pl.ds(rqP, qb), pl.ds(0, hc)], rP2, py)
        b = xchg(3, out_ref.at[pl.ds(rqQ, qb), pl.ds(hc, hc)], rQ2, px)
        a.wait()
        b.wait()
        out_ref[pl.ds(r0P, qb), pl.ds(0, hc)] = (
            out_ref[pl.ds(r0P, qb), pl.ds(0, hc)] + rP2[...]
        )
        out_ref[pl.ds(r0Q, qb), pl.ds(hc, hc)] = (
            out_ref[pl.ds(r0Q, qb), pl.ds(hc, hc)] + rQ2[...]
        )

        a = xchg(
            4,
            out_ref.at[pl.ds(r0P, qb), pl.ds(0, hc)],
            out_ref.at[pl.ds(r0P, qb), pl.ds(0, hc)],
            py,
        )
        b = xchg(
            5,
            out_ref.at[pl.ds(r0Q, qb), pl.ds(hc, hc)],
            out_ref.at[pl.ds(r0Q, qb), pl.ds(hc, hc)],
            px,
        )
        a.wait()
        b.wait()

        a = xchg(
            6,
            out_ref.at[pl.ds(bx, hb), pl.ds(0, hc)],
            out_ref.at[pl.ds(bx, hb), pl.ds(0, hc)],
            px,
        )
        b = xchg(
            7,
            out_ref.at[pl.ds(by, hb), pl.ds(hc, hc)],
            out_ref.at[pl.ds(by, hb), pl.ds(hc, hc)],
            py,
        )
        a.wait()
        b.wait()

    return pl.pallas_call(
        body,
        out_shape=jax.ShapeDtypeStruct((m, n), x.dtype),
        in_specs=[pl.BlockSpec(memory_space=pltpu.VMEM)],
        out_specs=pl.BlockSpec(memory_space=pltpu.VMEM),
        scratch_shapes=[
            pltpu.VMEM((hb, hc), x.dtype),
            pltpu.VMEM((qb, hc), x.dtype),
            pltpu.VMEM((hb, hc), x.dtype),
            pltpu.VMEM((qb, hc), x.dtype),
            pltpu.SemaphoreType.DMA((8,)),
            pltpu.SemaphoreType.DMA((8,)),
        ],
        compiler_params=pltpu.CompilerParams(collective_id=0),
    )(x)


# device time: 152978 ns/iter; 1.9515x vs baseline; 1.0267x over previous
import jax
import jax.numpy as jnp
from jax import lax
from jax.experimental import pallas as pl
from jax.experimental.pallas import tpu as pltpu

N_DEV = 4


def kernel(x):
    m, n = x.shape
    hc = n // 2
    hb = m // 2
    qb = m // 4

    def body(x_ref, out_ref, rP1, rP2, rQ1, rQ2, send_sems, recv_sems):
        my = lax.axis_index("i")
        xv = jnp.where(my >= 2, 1, 0)
        yv = jnp.where((my == 1) | (my == 2), 1, 0)
        px = 3 - my
        py = my ^ 1

        barrier_sem = pltpu.get_barrier_semaphore()
        for nbr in (px, py):
            pl.semaphore_signal(
                barrier_sem, inc=1,
                device_id=(nbr,), device_id_type=pl.DeviceIdType.MESH,
            )
        pl.semaphore_wait(barrier_sem, 2)

        bx = xv * hb
        bxp = (1 - xv) * hb
        fP = (1 - yv) * qb
        oP = yv * qb
        r0P = bx + oP
        rqP = bx + fP

        by = yv * hb
        byp = (1 - yv) * hb
        fQ = (1 - xv) * qb
        oQ = xv * qb
        r0Q = by + oQ
        rqQ = by + fQ

        cP = pl.ds(0, hc)
        cQ = pl.ds(hc, hc)

        def xchg(step, src, dst, peer):
            rdma = pltpu.make_async_remote_copy(
                src_ref=src,
                dst_ref=dst,
                send_sem=send_sems.at[step],
                recv_sem=recv_sems.at[step],
                device_id=(peer,),
                device_id_type=pl.DeviceIdType.MESH,
            )
            rdma.start()
            return rdma

        p1a = xchg(0, x_ref.at[pl.ds(bxp + fP, qb), cP], rP1.at[pl.ds(fP, qb), :], px)
        q1a = xchg(1, x_ref.at[pl.ds(byp + fQ, qb), cQ], rQ1.at[pl.ds(fQ, qb), :], py)
        p1b = xchg(2, x_ref.at[pl.ds(bxp + oP, qb), cP], rP1.at[pl.ds(oP, qb), :], px)
        q1b = xchg(3, x_ref.at[pl.ds(byp + oQ, qb), cQ], rQ1.at[pl.ds(oQ, qb), :], py)

        p1a.wait()
        q1a.wait()
        out_ref[pl.ds(rqP, qb), cP] = (
            x_ref[pl.ds(rqP, qb), cP] + rP1[pl.ds(fP, qb), :]
        )
        out_ref[pl.ds(rqQ, qb), cQ] = (
            x_ref[pl.ds(rqQ, qb), cQ] + rQ1[pl.ds(fQ, qb), :]
        )

        p2 = xchg(4, out_ref.at[pl.ds(rqP, qb), cP], rP2, py)
        q2 = xchg(5, out_ref.at[pl.ds(rqQ, qb), cQ], rQ2, px)

        p1b.wait()
        q1b.wait()
        out_ref[pl.ds(r0P, qb), cP] = (
            x_ref[pl.ds(r0P, qb), cP] + rP1[pl.ds(oP, qb), :]
        )
        out_ref[pl.ds(r0Q, qb), cQ] = (
            x_ref[pl.ds(r0Q, qb), cQ] + rQ1[pl.ds(oQ, qb), :]
        )

        p2.wait()
        q2.wait()
        out_ref[pl.ds(r0P, qb), cP] = out_ref[pl.ds(r0P, qb), cP] + rP2[...]
        out_ref[pl.ds(r0Q, qb), cQ] = out_ref[pl.ds(r0Q, qb), cQ] + rQ2[...]

        p3 = xchg(6, out_ref.at[pl.ds(r0P, qb), cP], out_ref.at[pl.ds(r0P, qb), cP], py)
        q3 = xchg(7, out_ref.at[pl.ds(r0Q, qb), cQ], out_ref.at[pl.ds(r0Q, qb), cQ], px)
        p4a = xchg(8, out_ref.at[pl.ds(r0P, qb), cP], out_ref.at[pl.ds(r0P, qb), cP], px)
        q4a = xchg(9, out_ref.at[pl.ds(r0Q, qb), cQ], out_ref.at[pl.ds(r0Q, qb), cQ], py)

        p3.wait()
        q3.wait()
        p4b = xchg(10, out_ref.at[pl.ds(rqP, qb), cP], out_ref.at[pl.ds(rqP, qb), cP], px)
        q4b = xchg(11, out_ref.at[pl.ds(rqQ, qb), cQ], out_ref.at[pl.ds(rqQ, qb), cQ], py)

        p4a.wait()
        q4a.wait()
        p4b.wait()
        q4b.wait()

    return pl.pallas_call(
        body,
        out_shape=jax.ShapeDtypeStruct((m, n), x.dtype),
        in_specs=[pl.BlockSpec(memory_space=pltpu.VMEM)],
        out_specs=pl.BlockSpec(memory_space=pltpu.VMEM),
        scratch_shapes=[
            pltpu.VMEM((hb, hc), x.dtype),
            pltpu.VMEM((qb, hc), x.dtype),
            pltpu.VMEM((hb, hc), x.dtype),
            pltpu.VMEM((qb, hc), x.dtype),
            pltpu.SemaphoreType.DMA((12,)),
            pltpu.SemaphoreType.DMA((12,)),
        ],
        compiler_params=pltpu.CompilerParams(collective_id=0),
    )(x)


# device time: 152913 ns/iter; 1.9523x vs baseline; 1.0004x over previous
import jax
import jax.numpy as jnp
from jax import lax
from jax.experimental import pallas as pl
from jax.experimental.pallas import tpu as pltpu

N_DEV = 4


def kernel(x):
    m, n = x.shape
    hm = m // 2
    hb = m // 4
    qb = m // 8

    def body(x_ref, out_ref, rP1, rP2, rQ1, rQ2, send_sems, recv_sems):
        my = lax.axis_index("i")
        xv = jnp.where(my >= 2, 1, 0)
        yv = jnp.where((my == 1) | (my == 2), 1, 0)
        px = 3 - my
        py = my ^ 1

        barrier_sem = pltpu.get_barrier_semaphore()
        for nbr in (px, py):
            pl.semaphore_signal(
                barrier_sem, inc=1,
                device_id=(nbr,), device_id_type=pl.DeviceIdType.MESH,
            )
        pl.semaphore_wait(barrier_sem, 2)

        bx = xv * hb
        bxp = (1 - xv) * hb
        fP = (1 - yv) * qb
        oP = yv * qb
        r0P = bx + oP
        rqP = bx + fP

        by = hm + yv * hb
        byp = hm + (1 - yv) * hb
        fQ = (1 - xv) * qb
        oQ = xv * qb
        r0Q = by + oQ
        rqQ = by + fQ

        def xchg(step, src, dst, peer):
            rdma = pltpu.make_async_remote_copy(
                src_ref=src,
                dst_ref=dst,
                send_sem=send_sems.at[step],
                recv_sem=recv_sems.at[step],
                device_id=(peer,),
                device_id_type=pl.DeviceIdType.MESH,
            )
            rdma.start()
            return rdma

        p1a = xchg(0, x_ref.at[pl.ds(bxp + fP, qb), :], rP1.at[pl.ds(fP, qb), :], px)
        q1a = xchg(1, x_ref.at[pl.ds(byp + fQ, qb), :], rQ1.at[pl.ds(fQ, qb), :], py)
        p1b = xchg(2, x_ref.at[pl.ds(bxp + oP, qb), :], rP1.at[pl.ds(oP, qb), :], px)
        q1b = xchg(3, x_ref.at[pl.ds(byp + oQ, qb), :], rQ1.at[pl.ds(oQ, qb), :], py)

        p1a.wait()
        q1a.wait()
        out_ref[pl.ds(rqP, qb), :] = (
            x_ref[pl.ds(rqP, qb), :] + rP1[pl.ds(fP, qb), :]
        )
        out_ref[pl.ds(rqQ, qb), :] = (
            x_ref[pl.ds(rqQ, qb), :] + rQ1[pl.ds(fQ, qb), :]
        )

        p2 = xchg(4, out_ref.at[pl.ds(rqP, qb), :], rP2, py)
        q2 = xchg(5, out_ref.at[pl.ds(rqQ, qb), :], rQ2, px)

        p1b.wait()
        q1b.wait()
        out_ref[pl.ds(r0P, qb), :] = (
            x_ref[pl.ds(r0P, qb), :] + rP1[pl.ds(oP, qb), :]
        )
        out_ref[pl.ds(r0Q, qb), :] = (
            x_ref[pl.ds(r0Q, qb), :] + rQ1[pl.ds(oQ, qb), :]
        )

        p2.wait()
        q2.wait()
        out_ref[pl.ds(r0P, qb), :] = out_ref[pl.ds(r0P, qb), :] + rP2[...]
        out_ref[pl.ds(r0Q, qb), :] = out_ref[pl.ds(r0Q, qb), :] + rQ2[...]

        p3 = xchg(6, out_ref.at[pl.ds(r0P, qb), :], out_ref.at[pl.ds(r0P, qb), :], py)
        q3 = xchg(7, out_ref.at[pl.ds(r0Q, qb), :], out_ref.at[pl.ds(r0Q, qb), :], px)
        p4a = xchg(8, out_ref.at[pl.ds(r0P, qb), :], out_ref.at[pl.ds(r0P, qb), :], px)
        q4a = xchg(9, out_ref.at[pl.ds(r0Q, qb), :], out_ref.at[pl.ds(r0Q, qb), :], py)

        p3.wait()
        q3.wait()
        p4b = xchg(10, out_ref.at[pl.ds(rqP, qb), :], out_ref.at[pl.ds(rqP, qb), :], px)
        q4b = xchg(11, out_ref.at[pl.ds(rqQ, qb), :], out_ref.at[pl.ds(rqQ, qb), :], py)

        p4a.wait()
        q4a.wait()
        p4b.wait()
        q4b.wait()

    return pl.pallas_call(
        body,
        out_shape=jax.ShapeDtypeStruct((m, n), x.dtype),
        in_specs=[pl.BlockSpec(memory_space=pltpu.VMEM)],
        out_specs=pl.BlockSpec(memory_space=pltpu.VMEM),
        scratch_shapes=[
            pltpu.VMEM((hb, n), x.dtype),
            pltpu.VMEM((qb, n), x.dtype),
            pltpu.VMEM((hb, n), x.dtype),
            pltpu.VMEM((qb, n), x.dtype),
            pltpu.SemaphoreType.DMA((12,)),
            pltpu.SemaphoreType.DMA((12,)),
        ],
        compiler_params=pltpu.CompilerParams(collective_id=0),
    )(x)
